# Initial kernel scaffold; baseline (speedup 1.0000x reference)
import jax
import jax.numpy as jnp
import numpy as np
from jax.experimental import pallas as pl

N_HEADS = 8


def _weight_kernel(a_ref, v0_ref, v1_ref, m0_ref, m1_ref):
    a = a_ref[...]
    v0 = v0_ref[...]
    v1 = v1_ref[...]
    a0 = jnp.repeat(a, 2, axis=1)
    a1 = jnp.repeat(a, 6, axis=1)
    m0_ref[...] = a0 * v0
    m1_ref[...] = a1 * v1


def kernel(v0, v1, k0, k1, q0, q1, edge_index):
    E = v0.shape[0]
    N = q0.shape[0]
    dst = edge_index[1]
    kh = jnp.concatenate([k0.reshape(E, N_HEADS, -1), k1.reshape(E, N_HEADS, -1)], axis=-1)
    qh = jnp.concatenate([q0.reshape(N, N_HEADS, -1), q1.reshape(N, N_HEADS, -1)], axis=-1)
    e = jnp.sum(kh * qh[dst], axis=-1) / np.sqrt(64.0)
    ee = jnp.exp(e)
    ssum = jax.ops.segment_sum(ee, dst, num_segments=N)
    a = ee / ssum[dst]

    B = 8000
    m0, m1 = pl.pallas_call(
        _weight_kernel,
        grid=(E // B,),
        in_specs=[
            pl.BlockSpec((B, 8), lambda i: (i, 0)),
            pl.BlockSpec((B, 16), lambda i: (i, 0)),
            pl.BlockSpec((B, 48), lambda i: (i, 0)),
        ],
        out_specs=[
            pl.BlockSpec((B, 16), lambda i: (i, 0)),
            pl.BlockSpec((B, 48), lambda i: (i, 0)),
        ],
        out_shape=[
            jax.ShapeDtypeStruct((E, 16), jnp.float32),
            jax.ShapeDtypeStruct((E, 48), jnp.float32),
        ],
    )(a, v0.reshape(E, 16), v1.reshape(E, 48))
    out0 = jax.ops.segment_sum(m0, dst, num_segments=N).reshape(N, 16, 1)
    out1 = jax.ops.segment_sum(m1, dst, num_segments=N).reshape(N, 16, 3)
    return out0, out1


# speedup vs baseline: 5.7525x; 5.7525x over previous
import jax
import jax.numpy as jnp
import numpy as np
from jax.experimental import pallas as pl

N_HEADS = 8


def _weight_kernel(a_ref, v0_ref, v1_ref, m0_ref, m1_ref):
    a = a_ref[...]
    B = a.shape[0]
    v0 = v0_ref[...].reshape(B, 8, 2)
    v1 = v1_ref[...].reshape(B, 8, 6)
    m0_ref[...] = (a[:, :, None] * v0).reshape(B, 16)
    m1_ref[...] = (a[:, :, None] * v1).reshape(B, 48)


def kernel(v0, v1, k0, k1, q0, q1, edge_index):
    E = v0.shape[0]
    N = q0.shape[0]
    dst = edge_index[1]
    kh = jnp.concatenate([k0.reshape(E, N_HEADS, -1), k1.reshape(E, N_HEADS, -1)], axis=-1)
    qh = jnp.concatenate([q0.reshape(N, N_HEADS, -1), q1.reshape(N, N_HEADS, -1)], axis=-1)
    e = jnp.sum(kh * qh[dst], axis=-1) / np.sqrt(64.0)
    ee = jnp.exp(e)
    ssum = jax.ops.segment_sum(ee, dst, num_segments=N)
    a = ee / ssum[dst]

    B = 2000
    m0, m1 = pl.pallas_call(
        _weight_kernel,
        grid=(E // B,),
        in_specs=[
            pl.BlockSpec((B, 8), lambda i: (i, 0)),
            pl.BlockSpec((B, 16), lambda i: (i, 0)),
            pl.BlockSpec((B, 48), lambda i: (i, 0)),
        ],
        out_specs=[
            pl.BlockSpec((B, 16), lambda i: (i, 0)),
            pl.BlockSpec((B, 48), lambda i: (i, 0)),
        ],
        out_shape=[
            jax.ShapeDtypeStruct((E, 16), jnp.float32),
            jax.ShapeDtypeStruct((E, 48), jnp.float32),
        ],
    )(a, v0.reshape(E, 16), v1.reshape(E, 48))
    out0 = jax.ops.segment_sum(m0, dst, num_segments=N).reshape(N, 16, 1)
    out1 = jax.ops.segment_sum(m1, dst, num_segments=N).reshape(N, 16, 3)
    return out0, out1


# speedup vs baseline: 34.9266x; 6.0715x over previous
import functools

import jax
import jax.numpy as jnp
from jax import lax
from jax.experimental import pallas as pl
from jax.experimental.pallas import tpu as pltpu
from jax.experimental.pallas import tpu_sc as plsc

N_NODES = 50000
N_EDGES = 800000
H = 8
C = 128
NCHUNK = N_EDGES // C
ZR = 128
NBLK = N_NODES // ZR
NTAIL = N_NODES - NBLK * ZR
GRP = C // 16

_MESH = plsc.VectorSubcoreMesh(core_axis_name="c", subcore_axis_name="s")


def _full(v):
    return jnp.full((16,), v, jnp.int32)


def _zero_vmem_rank2(ref, rows, cols):
    iota = lax.iota(jnp.int32, 16)
    z = jnp.zeros((16,), jnp.float32)
    for b in range(rows * cols // 16):
        p = iota + b * 16
        plsc.store_scatter(ref, [p // cols, p % cols], z)


def _zero_shared(z_v, acc_sh, sid, cols):
    _zero_vmem_rank2(z_v, ZR, cols)

    def body(ci, _):
        b = sid + ci * 16

        @pl.when(b < NBLK)
        def _():
            pltpu.sync_copy(z_v, acc_sh.at[pl.ds(b * ZR, ZR), :])
        return 0

    lax.fori_loop(0, (NBLK + 15) // 16, body, 0)

    @pl.when(sid == 15)
    def _():
        pltpu.sync_copy(z_v.at[pl.ds(0, NTAIL), :],
                        acc_sh.at[pl.ds(NBLK * ZR, NTAIL), :])


def _pass1_body(k0_hbm, k1_hbm, q0_hbm, q1_hbm, dst_hbm,
                ee_hbm, part0_hbm, part1_hbm,
                dst_v, k0_v, k1_v, q0_v, q1_v, ee_v, z_v, acc_sh,
                sem0, sem1):
    cid = lax.axis_index("c")
    sid = lax.axis_index("s")
    wid = cid * 16 + sid

    _zero_shared(z_v, acc_sh, sid, 8)
    plsc.subcore_barrier()

    iota = lax.iota(jnp.int32, 16)

    def chunk_body(ci, _):
        chunk = wid + ci * 32

        @pl.when(chunk < NCHUNK)
        def _():
            off = chunk * C
            pltpu.sync_copy(dst_hbm.at[pl.ds(off, C)], dst_v)
            pltpu.sync_copy(k0_hbm.at[pl.ds(off, C), :], k0_v)
            pltpu.sync_copy(k1_hbm.at[pl.ds(off, C), :], k1_v)
            cp0 = pltpu.async_copy(q0_hbm.at[dst_v], q0_v, sem0)
            cp1 = pltpu.async_copy(q1_hbm.at[dst_v], q1_v, sem1)
            cp0.wait()
            cp1.wait()
            for g in range(GRP):
                rows = iota + g * 16
                for h in range(H):
                    acc = jnp.zeros((16,), jnp.float32)
                    for t in range(2):
                        j = 2 * h + t
                        acc += (plsc.load_gather(k0_v, [rows, _full(j)])
                                * plsc.load_gather(q0_v, [rows, _full(j)]))
                    for t in range(6):
                        j = 6 * h + t
                        acc += (plsc.load_gather(k1_v, [rows, _full(j)])
                                * plsc.load_gather(q1_v, [rows, _full(j)]))
                    ee = jnp.exp(acc * 0.125)
                    plsc.store_scatter(ee_v, [rows, _full(h)], ee)
            pltpu.sync_copy(ee_v, ee_hbm.at[pl.ds(off, C), :])
            pltpu.sync_copy(ee_v, acc_sh.at[dst_v], add=True)
        return 0

    lax.fori_loop(0, (NCHUNK + 31) // 32, chunk_body, 0)
    plsc.subcore_barrier()

    def out_body(ci, _):
        b = sid + ci * 16

        @pl.when(b < NBLK)
        def _():
            src = acc_sh.at[pl.ds(b * ZR, ZR), :]

            @pl.when(cid == 0)
            def _():
                pltpu.sync_copy(src, part0_hbm.at[pl.ds(b * ZR, ZR), :])

            @pl.when(cid == 1)
            def _():
                pltpu.sync_copy(src, part1_hbm.at[pl.ds(b * ZR, ZR), :])
        return 0

    lax.fori_loop(0, (NBLK + 15) // 16, out_body, 0)

    @pl.when(sid == 15)
    def _():
        src = acc_sh.at[pl.ds(NBLK * ZR, NTAIL), :]

        @pl.when(cid == 0)
        def _():
            pltpu.sync_copy(src, part0_hbm.at[pl.ds(NBLK * ZR, NTAIL), :])

        @pl.when(cid == 1)
        def _():
            pltpu.sync_copy(src, part1_hbm.at[pl.ds(NBLK * ZR, NTAIL), :])


_pass1 = functools.partial(
    pl.kernel,
    out_type=(jax.ShapeDtypeStruct((N_EDGES, 8), jnp.float32),
              jax.ShapeDtypeStruct((N_NODES, 8), jnp.float32),
              jax.ShapeDtypeStruct((N_NODES, 8), jnp.float32)),
    mesh=_MESH,
    compiler_params=pltpu.CompilerParams(
        use_tc_tiling_on_sc=False, needs_layout_passes=False),
    scratch_types=[
        pltpu.VMEM((C,), jnp.int32),
        pltpu.VMEM((C, 16), jnp.float32),
        pltpu.VMEM((C, 48), jnp.float32),
        pltpu.VMEM((C, 16), jnp.float32),
        pltpu.VMEM((C, 48), jnp.float32),
        pltpu.VMEM((C, 8), jnp.float32),
        pltpu.VMEM((ZR, 8), jnp.float32),
        pltpu.VMEM_SHARED((N_NODES, 8), jnp.float32),
        pltpu.SemaphoreType.DMA,
        pltpu.SemaphoreType.DMA,
    ],
)(_pass1_body)


def _merge_body(p0_ref, p1_ref, out_ref):
    out_ref[...] = p0_ref[...] + p1_ref[...]


def _merge(p0, p1):
    r = pl.pallas_call(
        _merge_body,
        out_shape=jax.ShapeDtypeStruct((N_NODES * 8 // 128, 128), jnp.float32),
    )(p0.reshape(-1, 128), p1.reshape(-1, 128))
    return r.reshape(N_NODES, 8)


_HEADS_A0 = [c // 2 for c in range(16)]
_HEADS_B0 = [c // 6 for c in range(16)]
_HEADS_A1 = [(16 + c) // 6 for c in range(16)]
_HEADS_B1 = [(32 + c) // 6 for c in range(16)]


def _pass2_compute(rows, ee_v, s_v, va_v, vb_v, m_v, heads_a, heads_b):
    used = sorted(set(heads_a) | set(heads_b))
    ah = {}
    for h in used:
        ah[h] = (plsc.load_gather(ee_v, [rows, _full(h)])
                 / plsc.load_gather(s_v, [rows, _full(h)]))
    for c in range(16):
        mv = ah[heads_a[c]] * plsc.load_gather(va_v, [rows, _full(c)])
        plsc.store_scatter(m_v, [rows, _full(c)], mv)
    for c in range(16):
        mv = ah[heads_b[c]] * plsc.load_gather(vb_v, [rows, _full(c)])
        plsc.store_scatter(m_v, [rows, _full(16 + c)], mv)


def _pass2_body(v0_hbm, v1_hbm, ee_hbm, ssum_hbm, dst_hbm,
                out0_hbm, out1_hbm,
                dst_v, ee_v, s_v, va_v, vb_v, m_v, z_v, acc_sh, sem0):
    cid = lax.axis_index("c")
    sid = lax.axis_index("s")

    _zero_shared(z_v, acc_sh, sid, 32)
    plsc.subcore_barrier()

    iota = lax.iota(jnp.int32, 16)

    def chunk_body(ci, _):
        chunk = sid + ci * 16

        @pl.when(chunk < NCHUNK)
        def _():
            off = chunk * C
            pltpu.sync_copy(dst_hbm.at[pl.ds(off, C)], dst_v)
            pltpu.sync_copy(ee_hbm.at[pl.ds(off, C), :], ee_v)
            cp = pltpu.async_copy(ssum_hbm.at[dst_v], s_v, sem0)

            @pl.when(cid == 0)
            def _():
                pltpu.sync_copy(v0_hbm.at[pl.ds(off, C), :], va_v)
                pltpu.sync_copy(v1_hbm.at[pl.ds(off, C), pl.ds(0, 16)], vb_v)

            @pl.when(cid == 1)
            def _():
                pltpu.sync_copy(v1_hbm.at[pl.ds(off, C), pl.ds(16, 16)], va_v)
                pltpu.sync_copy(v1_hbm.at[pl.ds(off, C), pl.ds(32, 16)], vb_v)

            cp.wait()

            @pl.when(cid == 0)
            def _():
                for g in range(GRP):
                    _pass2_compute(iota + g * 16, ee_v, s_v, va_v, vb_v, m_v,
                                   _HEADS_A0, _HEADS_B0)

            @pl.when(cid == 1)
            def _():
                for g in range(GRP):
                    _pass2_compute(iota + g * 16, ee_v, s_v, va_v, vb_v, m_v,
                                   _HEADS_A1, _HEADS_B1)

            pltpu.sync_copy(m_v, acc_sh.at[dst_v], add=True)
        return 0

    lax.fori_loop(0, (NCHUNK + 15) // 16, chunk_body, 0)
    plsc.subcore_barrier()

    def out_body(ci, _):
        b = sid + ci * 16

        @pl.when(b < NBLK)
        def _():
            _copy_out_rows(acc_sh, out0_hbm, out1_hbm, cid, b * ZR, ZR)
        return 0

    lax.fori_loop(0, (NBLK + 15) // 16, out_body, 0)

    @pl.when(sid == 15)
    def _():
        _copy_out_rows(acc_sh, out0_hbm, out1_hbm, cid, NBLK * ZR, NTAIL)


def _copy_out_rows(acc_sh, out0_hbm, out1_hbm, cid, r0, nr):
    rows = pl.ds(r0, nr)

    @pl.when(cid == 0)
    def _():
        pltpu.sync_copy(acc_sh.at[rows, pl.ds(0, 16)], out0_hbm.at[rows, :])
        pltpu.sync_copy(acc_sh.at[rows, pl.ds(16, 16)],
                        out1_hbm.at[rows, pl.ds(0, 16)])

    @pl.when(cid == 1)
    def _():
        pltpu.sync_copy(acc_sh.at[rows, pl.ds(0, 16)],
                        out1_hbm.at[rows, pl.ds(16, 16)])
        pltpu.sync_copy(acc_sh.at[rows, pl.ds(16, 16)],
                        out1_hbm.at[rows, pl.ds(32, 16)])


_pass2 = functools.partial(
    pl.kernel,
    out_type=(jax.ShapeDtypeStruct((N_NODES, 16), jnp.float32),
              jax.ShapeDtypeStruct((N_NODES, 48), jnp.float32)),
    mesh=_MESH,
    compiler_params=pltpu.CompilerParams(
        use_tc_tiling_on_sc=False, needs_layout_passes=False),
    scratch_types=[
        pltpu.VMEM((C,), jnp.int32),
        pltpu.VMEM((C, 8), jnp.float32),
        pltpu.VMEM((C, 8), jnp.float32),
        pltpu.VMEM((C, 16), jnp.float32),
        pltpu.VMEM((C, 16), jnp.float32),
        pltpu.VMEM((C, 32), jnp.float32),
        pltpu.VMEM((ZR, 32), jnp.float32),
        pltpu.VMEM_SHARED((N_NODES, 32), jnp.float32),
        pltpu.SemaphoreType.DMA,
    ],
)(_pass2_body)


def kernel(v0, v1, k0, k1, q0, q1, edge_index):
    E, N = N_EDGES, N_NODES
    dst = edge_index[1]
    k0f = k0.reshape(E, 16)
    k1f = k1.reshape(E, 48)
    q0f = q0.reshape(N, 16)
    q1f = q1.reshape(N, 48)
    v0f = v0.reshape(E, 16)
    v1f = v1.reshape(E, 48)

    ee, part0, part1 = _pass1(k0f, k1f, q0f, q1f, dst)
    ssum = _merge(part0, part1)
    out0f, out1f = _pass2(v0f, v1f, ee, ssum, dst)
    return out0f.reshape(N, 16, 1), out1f.reshape(N, 16, 3)


# speedup vs baseline: 40.3237x; 1.1545x over previous
import functools

import jax
import jax.numpy as jnp
from jax import lax
from jax.experimental import pallas as pl
from jax.experimental.pallas import tpu as pltpu
from jax.experimental.pallas import tpu_sc as plsc

N_NODES = 50000
N_EDGES = 800000
H = 8
C = 128
NCHUNK = N_EDGES // C
ZR = 128
NBLK = N_NODES // ZR
NTAIL = N_NODES - NBLK * ZR
GRP = C // 16

_MESH = plsc.VectorSubcoreMesh(core_axis_name="c", subcore_axis_name="s")
_PARAMS = pltpu.CompilerParams(use_tc_tiling_on_sc=False,
                               needs_layout_passes=False)


def _full(v):
    return jnp.full((16,), v, jnp.int32)


def _zero_vmem_rank2(ref, rows, cols):
    iota = lax.iota(jnp.int32, 16)
    z = jnp.zeros((16,), jnp.float32)
    for b in range(rows * cols // 16):
        p = iota + b * 16
        plsc.store_scatter(ref, [p // cols, p % cols], z)


def _zero_shared(z_v, acc_sh, sid, cols):
    _zero_vmem_rank2(z_v, ZR, cols)

    def body(ci, _):
        b = sid + ci * 16

        @pl.when(b < NBLK)
        def _():
            pltpu.sync_copy(z_v, acc_sh.at[pl.ds(b * ZR, ZR), :])
        return 0

    lax.fori_loop(0, (NBLK + 15) // 16, body, 0)

    @pl.when(sid == 15)
    def _():
        pltpu.sync_copy(z_v.at[pl.ds(0, NTAIL), :],
                        acc_sh.at[pl.ds(NBLK * ZR, NTAIL), :])


def _pass1_body(k0_hbm, k1_hbm, q0_hbm, q1_hbm, dst_hbm,
                ee_hbm, part0_hbm, part1_hbm,
                dst_v0, dst_v1, k0_v0, k0_v1, k1_v0, k1_v1,
                q0_v0, q0_v1, q1_v0, q1_v1, ee_v0, ee_v1, z_v, acc_sh,
                semD0, semD1, semL0, semL1, semG0, semG1, semO0, semO1):
    dst_v = [dst_v0, dst_v1]
    k0_v = [k0_v0, k0_v1]
    k1_v = [k1_v0, k1_v1]
    q0_v = [q0_v0, q0_v1]
    q1_v = [q1_v0, q1_v1]
    ee_v = [ee_v0, ee_v1]
    semD = [semD0, semD1]
    semL = [semL0, semL1]
    semG = [semG0, semG1]
    semO = [semO0, semO1]

    cid = lax.axis_index("c")
    sid = lax.axis_index("s")
    wid = cid * 16 + sid

    _zero_shared(z_v, acc_sh, sid, 8)
    plsc.subcore_barrier()

    iota = lax.iota(jnp.int32, 16)

    def issue_loads(chunk, s):
        off = chunk * C
        pltpu.async_copy(k0_hbm.at[pl.ds(off, C), :], k0_v[s], semL[s])
        pltpu.async_copy(k1_hbm.at[pl.ds(off, C), :], k1_v[s], semL[s])

    def wait_loads(s):
        pltpu.make_async_copy(k0_hbm.at[pl.ds(0, C), :], k0_v[s], semL[s]).wait()
        pltpu.make_async_copy(k1_hbm.at[pl.ds(0, C), :], k1_v[s], semL[s]).wait()

    def do_gathers(s):
        cg0 = pltpu.async_copy(q0_hbm.at[dst_v[s]], q0_v[s], semG[s])
        cg1 = pltpu.async_copy(q1_hbm.at[dst_v[s]], q1_v[s], semG[s])
        cg0.wait()
        cg1.wait()

    def drain_outputs(s):
        pltpu.make_async_copy(ee_v[s], ee_hbm.at[pl.ds(0, C), :], semO[s]).wait()

    def compute(s):
        for g in range(GRP):
            rows = iota + g * 16
            for h in range(H):
                acc = jnp.zeros((16,), jnp.float32)
                for t in range(2):
                    j = 2 * h + t
                    acc += (plsc.load_gather(k0_v[s], [rows, _full(j)])
                            * plsc.load_gather(q0_v[s], [rows, _full(j)]))
                for t in range(6):
                    j = 6 * h + t
                    acc += (plsc.load_gather(k1_v[s], [rows, _full(j)])
                            * plsc.load_gather(q1_v[s], [rows, _full(j)]))
                ee = jnp.exp(acc * 0.125)
                plsc.store_scatter(ee_v[s], [rows, _full(h)], ee)

    pltpu.sync_copy(dst_hbm.at[pl.ds(wid * C, C)], dst_v[0])
    issue_loads(wid, 0)

    def pair_body(cp, _):
        for b in (0, 1):
            chunk = wid + (2 * cp + b) * 32
            nchunk = chunk + 32

            @pl.when(chunk < NCHUNK)
            def _():
                if b == 1:
                    drain_outputs(0)
                else:
                    @pl.when(cp >= 1)
                    def _():
                        drain_outputs(1)

                @pl.when(nchunk < NCHUNK)
                def _():
                    pltpu.async_copy(dst_hbm.at[pl.ds(nchunk * C, C)],
                                     dst_v[1 - b], semD[1 - b])

                do_gathers(b)
                wait_loads(b)
                compute(b)
                pltpu.async_copy(ee_v[b], ee_hbm.at[pl.ds(chunk * C, C), :],
                                 semO[b])
                pltpu.sync_copy(ee_v[b], acc_sh.at[dst_v[b]], add=True)

                @pl.when(nchunk < NCHUNK)
                def _():
                    pltpu.make_async_copy(dst_hbm.at[pl.ds(0, C)],
                                          dst_v[1 - b], semD[1 - b]).wait()
                    issue_loads(nchunk, 1 - b)
        return 0

    npairs = ((NCHUNK + 31) // 32 + 1) // 2
    lax.fori_loop(0, npairs, pair_body, 0)

    nct = (NCHUNK - wid + 31) // 32
    last = (nct - 1) % 2

    @pl.when(last == 0)
    def _():
        drain_outputs(0)

    @pl.when(last == 1)
    def _():
        drain_outputs(1)

    plsc.subcore_barrier()

    def out_body(ci, _):
        b = sid + ci * 16

        @pl.when(b < NBLK)
        def _():
            src = acc_sh.at[pl.ds(b * ZR, ZR), :]

            @pl.when(cid == 0)
            def _():
                pltpu.sync_copy(src, part0_hbm.at[pl.ds(b * ZR, ZR), :])

            @pl.when(cid == 1)
            def _():
                pltpu.sync_copy(src, part1_hbm.at[pl.ds(b * ZR, ZR), :])
        return 0

    lax.fori_loop(0, (NBLK + 15) // 16, out_body, 0)

    @pl.when(sid == 15)
    def _():
        src = acc_sh.at[pl.ds(NBLK * ZR, NTAIL), :]

        @pl.when(cid == 0)
        def _():
            pltpu.sync_copy(src, part0_hbm.at[pl.ds(NBLK * ZR, NTAIL), :])

        @pl.when(cid == 1)
        def _():
            pltpu.sync_copy(src, part1_hbm.at[pl.ds(NBLK * ZR, NTAIL), :])


_pass1 = functools.partial(
    pl.kernel,
    out_type=(jax.ShapeDtypeStruct((N_EDGES, 8), jnp.float32),
              jax.ShapeDtypeStruct((N_NODES, 8), jnp.float32),
              jax.ShapeDtypeStruct((N_NODES, 8), jnp.float32)),
    mesh=_MESH,
    compiler_params=_PARAMS,
    scratch_types=[
        pltpu.VMEM((C,), jnp.int32),
        pltpu.VMEM((C,), jnp.int32),
        pltpu.VMEM((C, 16), jnp.float32),
        pltpu.VMEM((C, 16), jnp.float32),
        pltpu.VMEM((C, 48), jnp.float32),
        pltpu.VMEM((C, 48), jnp.float32),
        pltpu.VMEM((C, 16), jnp.float32),
        pltpu.VMEM((C, 16), jnp.float32),
        pltpu.VMEM((C, 48), jnp.float32),
        pltpu.VMEM((C, 48), jnp.float32),
        pltpu.VMEM((C, 8), jnp.float32),
        pltpu.VMEM((C, 8), jnp.float32),
        pltpu.VMEM((ZR, 8), jnp.float32),
        pltpu.VMEM_SHARED((N_NODES, 8), jnp.float32),
        pltpu.SemaphoreType.DMA,
        pltpu.SemaphoreType.DMA,
        pltpu.SemaphoreType.DMA,
        pltpu.SemaphoreType.DMA,
        pltpu.SemaphoreType.DMA,
        pltpu.SemaphoreType.DMA,
        pltpu.SemaphoreType.DMA,
        pltpu.SemaphoreType.DMA,
    ],
)(_pass1_body)


def _merge_body(p0_ref, p1_ref, out_ref):
    out_ref[...] = p0_ref[...] + p1_ref[...]


def _merge(p0, p1):
    r = pl.pallas_call(
        _merge_body,
        out_shape=jax.ShapeDtypeStruct((N_NODES * 8 // 128, 128), jnp.float32),
    )(p0.reshape(-1, 128), p1.reshape(-1, 128))
    return r.reshape(N_NODES, 8)


_HEADS_A0 = [c // 2 for c in range(16)]
_HEADS_B0 = [c // 6 for c in range(16)]
_HEADS_A1 = [(16 + c) // 6 for c in range(16)]
_HEADS_B1 = [(32 + c) // 6 for c in range(16)]


def _pass2_compute(rows, ee_v, s_v, va_v, vb_v, m_v, heads_a, heads_b):
    used = sorted(set(heads_a) | set(heads_b))
    ah = {}
    for h in used:
        ah[h] = (plsc.load_gather(ee_v, [rows, _full(h)])
                 / plsc.load_gather(s_v, [rows, _full(h)]))
    for c in range(16):
        mv = ah[heads_a[c]] * plsc.load_gather(va_v, [rows, _full(c)])
        plsc.store_scatter(m_v, [rows, _full(c)], mv)
    for c in range(16):
        mv = ah[heads_b[c]] * plsc.load_gather(vb_v, [rows, _full(c)])
        plsc.store_scatter(m_v, [rows, _full(16 + c)], mv)


def _pass2_body(v0_hbm, v1_hbm, ee_hbm, ssum_hbm, dst_hbm,
                out0_hbm, out1_hbm,
                dst_v0, dst_v1, ee_v0, ee_v1, s_v0, s_v1,
                va_v0, va_v1, vb_v0, vb_v1, m_v0, m_v1, z_v, acc_sh,
                semD0, semD1, semL0, semL1, semG0, semG1, semO0, semO1):
    dst_v = [dst_v0, dst_v1]
    ee_v = [ee_v0, ee_v1]
    s_v = [s_v0, s_v1]
    va_v = [va_v0, va_v1]
    vb_v = [vb_v0, vb_v1]
    m_v = [m_v0, m_v1]
    semD = [semD0, semD1]
    semL = [semL0, semL1]
    semG = [semG0, semG1]
    semO = [semO0, semO1]

    cid = lax.axis_index("c")
    sid = lax.axis_index("s")

    _zero_shared(z_v, acc_sh, sid, 32)
    plsc.subcore_barrier()

    iota = lax.iota(jnp.int32, 16)

    def issue_loads(chunk, s):
        off = chunk * C
        pltpu.async_copy(ee_hbm.at[pl.ds(off, C), :], ee_v[s], semL[s])

        @pl.when(cid == 0)
        def _():
            pltpu.async_copy(v0_hbm.at[pl.ds(off, C), :], va_v[s], semL[s])
            pltpu.async_copy(v1_hbm.at[pl.ds(off, C), pl.ds(0, 16)], vb_v[s],
                             semL[s])

        @pl.when(cid == 1)
        def _():
            pltpu.async_copy(v1_hbm.at[pl.ds(off, C), pl.ds(16, 16)], va_v[s],
                             semL[s])
            pltpu.async_copy(v1_hbm.at[pl.ds(off, C), pl.ds(32, 16)], vb_v[s],
                             semL[s])

    def wait_loads(s):
        pltpu.make_async_copy(ee_hbm.at[pl.ds(0, C), :], ee_v[s], semL[s]).wait()
        pltpu.make_async_copy(v0_hbm.at[pl.ds(0, C), :], va_v[s], semL[s]).wait()
        pltpu.make_async_copy(v1_hbm.at[pl.ds(0, C), pl.ds(0, 16)], vb_v[s],
                              semL[s]).wait()

    def do_gathers(s):
        pltpu.async_copy(ssum_hbm.at[dst_v[s]], s_v[s], semG[s]).wait()

    def compute(s):
        @pl.when(cid == 0)
        def _():
            for g in range(GRP):
                _pass2_compute(iota + g * 16, ee_v[s], s_v[s], va_v[s],
                               vb_v[s], m_v[s], _HEADS_A0, _HEADS_B0)

        @pl.when(cid == 1)
        def _():
            for g in range(GRP):
                _pass2_compute(iota + g * 16, ee_v[s], s_v[s], va_v[s],
                               vb_v[s], m_v[s], _HEADS_A1, _HEADS_B1)

    pltpu.sync_copy(dst_hbm.at[pl.ds(sid * C, C)], dst_v[0])
    issue_loads(sid, 0)

    def pair_body(cp, _):
        for b in (0, 1):
            chunk = sid + (2 * cp + b) * 16
            nchunk = chunk + 16

            @pl.when(chunk < NCHUNK)
            def _():
                @pl.when(nchunk < NCHUNK)
                def _():
                    pltpu.async_copy(dst_hbm.at[pl.ds(nchunk * C, C)],
                                     dst_v[1 - b], semD[1 - b])

                do_gathers(b)
                wait_loads(b)
                compute(b)
                pltpu.sync_copy(m_v[b], acc_sh.at[dst_v[b]], add=True)

                @pl.when(nchunk < NCHUNK)
                def _():
                    pltpu.make_async_copy(dst_hbm.at[pl.ds(0, C)],
                                          dst_v[1 - b], semD[1 - b]).wait()
                    issue_loads(nchunk, 1 - b)
        return 0

    npairs = ((NCHUNK + 15) // 16 + 1) // 2
    lax.fori_loop(0, npairs, pair_body, 0)

    plsc.subcore_barrier()

    def out_body(ci, _):
        b = sid + ci * 16

        @pl.when(b < NBLK)
        def _():
            _copy_out_rows(acc_sh, out0_hbm, out1_hbm, cid, b * ZR, ZR)
        return 0

    lax.fori_loop(0, (NBLK + 15) // 16, out_body, 0)

    @pl.when(sid == 15)
    def _():
        _copy_out_rows(acc_sh, out0_hbm, out1_hbm, cid, NBLK * ZR, NTAIL)


def _copy_out_rows(acc_sh, out0_hbm, out1_hbm, cid, r0, nr):
    rows = pl.ds(r0, nr)

    @pl.when(cid == 0)
    def _():
        pltpu.sync_copy(acc_sh.at[rows, pl.ds(0, 16)], out0_hbm.at[rows, :])
        pltpu.sync_copy(acc_sh.at[rows, pl.ds(16, 16)],
                        out1_hbm.at[rows, pl.ds(0, 16)])

    @pl.when(cid == 1)
    def _():
        pltpu.sync_copy(acc_sh.at[rows, pl.ds(0, 16)],
                        out1_hbm.at[rows, pl.ds(16, 16)])
        pltpu.sync_copy(acc_sh.at[rows, pl.ds(16, 16)],
                        out1_hbm.at[rows, pl.ds(32, 16)])


_pass2 = functools.partial(
    pl.kernel,
    out_type=(jax.ShapeDtypeStruct((N_NODES, 16), jnp.float32),
              jax.ShapeDtypeStruct((N_NODES, 48), jnp.float32)),
    mesh=_MESH,
    compiler_params=_PARAMS,
    scratch_types=[
        pltpu.VMEM((C,), jnp.int32),
        pltpu.VMEM((C,), jnp.int32),
        pltpu.VMEM((C, 8), jnp.float32),
        pltpu.VMEM((C, 8), jnp.float32),
        pltpu.VMEM((C, 8), jnp.float32),
        pltpu.VMEM((C, 8), jnp.float32),
        pltpu.VMEM((C, 16), jnp.float32),
        pltpu.VMEM((C, 16), jnp.float32),
        pltpu.VMEM((C, 16), jnp.float32),
        pltpu.VMEM((C, 16), jnp.float32),
        pltpu.VMEM((C, 32), jnp.float32),
        pltpu.VMEM((C, 32), jnp.float32),
        pltpu.VMEM((ZR, 32), jnp.float32),
        pltpu.VMEM_SHARED((N_NODES, 32), jnp.float32),
        pltpu.SemaphoreType.DMA,
        pltpu.SemaphoreType.DMA,
        pltpu.SemaphoreType.DMA,
        pltpu.SemaphoreType.DMA,
        pltpu.SemaphoreType.DMA,
        pltpu.SemaphoreType.DMA,
        pltpu.SemaphoreType.DMA,
        pltpu.SemaphoreType.DMA,
    ],
)(_pass2_body)


def kernel(v0, v1, k0, k1, q0, q1, edge_index):
    E, N = N_EDGES, N_NODES
    dst = edge_index[1]
    k0f = k0.reshape(E, 16)
    k1f = k1.reshape(E, 48)
    q0f = q0.reshape(N, 16)
    q1f = q1.reshape(N, 48)
    v0f = v0.reshape(E, 16)
    v1f = v1.reshape(E, 48)

    ee, part0, part1 = _pass1(k0f, k1f, q0f, q1f, dst)
    ssum = _merge(part0, part1)
    out0f, out1f = _pass2(v0f, v1f, ee, ssum, dst)
    return out0f.reshape(N, 16, 1), out1f.reshape(N, 16, 3)


# speedup vs baseline: 43.2618x; 1.0729x over previous
import functools

import jax
import jax.numpy as jnp
from jax import lax
from jax.experimental import pallas as pl
from jax.experimental.pallas import tpu as pltpu
from jax.experimental.pallas import tpu_sc as plsc

N_NODES = 50000
N_EDGES = 800000
H = 8
C = 256
CH = 128
NCHUNK = N_EDGES // C
ZR = 128
NBLK = N_NODES // ZR
NTAIL = N_NODES - NBLK * ZR
GRP = C // 16

_MESH = plsc.VectorSubcoreMesh(core_axis_name="c", subcore_axis_name="s")
_PARAMS = pltpu.CompilerParams(use_tc_tiling_on_sc=False,
                               needs_layout_passes=False)


def _full(v):
    return jnp.full((16,), v, jnp.int32)


def _zero_vmem_rank2(ref, rows, cols):
    iota = lax.iota(jnp.int32, 16)
    z = jnp.zeros((16,), jnp.float32)
    for b in range(rows * cols // 16):
        p = iota + b * 16
        plsc.store_scatter(ref, [p // cols, p % cols], z)


def _zero_shared(z_v, acc_sh, sid, cols, zr):
    _zero_vmem_rank2(z_v, zr, cols)
    nblk = N_NODES // zr
    ntail = N_NODES - nblk * zr

    def body(ci, _):
        b = sid + ci * 16

        @pl.when(b < nblk)
        def _():
            pltpu.sync_copy(z_v, acc_sh.at[pl.ds(b * zr, zr), :])
        return 0

    lax.fori_loop(0, (nblk + 15) // 16, body, 0)

    if ntail:
        @pl.when(sid == 15)
        def _():
            pltpu.sync_copy(z_v.at[pl.ds(0, ntail), :],
                            acc_sh.at[pl.ds(nblk * zr, ntail), :])


def _pass1_body(k0_hbm, k1_hbm, q0_hbm, q1_hbm, dst_hbm,
                ee_hbm, part0_hbm, part1_hbm,
                dst_v0, dst_v1, k0_v0, k0_v1, k1_v0, k1_v1,
                q0_v0, q0_v1, q1_v0, q1_v1, ee_v0, ee_v1, z_v, acc_sh,
                semD0, semD1, semL0, semL1, semG0, semG1, semO0, semO1):
    dst_v = [dst_v0, dst_v1]
    k0_v = [k0_v0, k0_v1]
    k1_v = [k1_v0, k1_v1]
    q0_v = [q0_v0, q0_v1]
    q1_v = [q1_v0, q1_v1]
    ee_v = [ee_v0, ee_v1]
    semD = [semD0, semD1]
    semL = [semL0, semL1]
    semG = [semG0, semG1]
    semO = [semO0, semO1]

    cid = lax.axis_index("c")
    sid = lax.axis_index("s")
    wid = cid * 16 + sid

    _zero_shared(z_v, acc_sh, sid, 8, ZR)
    plsc.subcore_barrier()

    iota = lax.iota(jnp.int32, 16)

    def issue_loads(chunk, s):
        off = chunk * C
        pltpu.async_copy(k0_hbm.at[pl.ds(off, C), :], k0_v[s], semL[s])
        pltpu.async_copy(k1_hbm.at[pl.ds(off, C), :], k1_v[s], semL[s])

    def wait_loads(s):
        pltpu.make_async_copy(k0_hbm.at[pl.ds(0, C), :], k0_v[s], semL[s]).wait()
        pltpu.make_async_copy(k1_hbm.at[pl.ds(0, C), :], k1_v[s], semL[s]).wait()

    def do_gathers(s):
        cps = []
        for hh in range(2):
            rr = pl.ds(hh * CH, CH)
            cps.append(pltpu.async_copy(q0_hbm.at[dst_v[s].at[hh]],
                                        q0_v[s].at[rr, :], semG[s]))
            cps.append(pltpu.async_copy(q1_hbm.at[dst_v[s].at[hh]],
                                        q1_v[s].at[rr, :], semG[s]))
        for cp in cps:
            cp.wait()

    def drain_outputs(s):
        pltpu.make_async_copy(ee_v[s], ee_hbm.at[pl.ds(0, C), :], semO[s]).wait()

    def compute(s):
        def gbody(g, _):
            rows = iota + g * 16
            for h in range(H):
                acc = jnp.zeros((16,), jnp.float32)
                for t in range(2):
                    j = 2 * h + t
                    acc += (plsc.load_gather(k0_v[s], [rows, _full(j)])
                            * plsc.load_gather(q0_v[s], [rows, _full(j)]))
                for t in range(6):
                    j = 6 * h + t
                    acc += (plsc.load_gather(k1_v[s], [rows, _full(j)])
                            * plsc.load_gather(q1_v[s], [rows, _full(j)]))
                ee = jnp.exp(acc * 0.125)
                plsc.store_scatter(ee_v[s], [rows, _full(h)], ee)
            return 0

        lax.fori_loop(0, GRP, gbody, 0)

    pltpu.sync_copy(dst_hbm.at[pl.ds(wid * 2, 2), :], dst_v[0])
    issue_loads(wid, 0)

    def pair_body(cp, _):
        for b in (0, 1):
            chunk = wid + (2 * cp + b) * 32
            nchunk = chunk + 32

            @pl.when(chunk < NCHUNK)
            def _():
                if b == 1:
                    drain_outputs(0)
                else:
                    @pl.when(cp >= 1)
                    def _():
                        drain_outputs(1)

                @pl.when(nchunk < NCHUNK)
                def _():
                    pltpu.async_copy(dst_hbm.at[pl.ds(nchunk * 2, 2), :],
                                     dst_v[1 - b], semD[1 - b])

                do_gathers(b)
                wait_loads(b)
                compute(b)
                pltpu.async_copy(ee_v[b], ee_hbm.at[pl.ds(chunk * C, C), :],
                                 semO[b])
                pltpu.sync_copy(ee_v[b].at[pl.ds(0, CH), :],
                                acc_sh.at[dst_v[b].at[0]], add=True)
                pltpu.sync_copy(ee_v[b].at[pl.ds(CH, CH), :],
                                acc_sh.at[dst_v[b].at[1]], add=True)

                @pl.when(nchunk < NCHUNK)
                def _():
                    pltpu.make_async_copy(dst_hbm.at[pl.ds(0, 2), :],
                                          dst_v[1 - b], semD[1 - b]).wait()
                    issue_loads(nchunk, 1 - b)
        return 0

    npairs = ((NCHUNK + 31) // 32 + 1) // 2
    lax.fori_loop(0, npairs, pair_body, 0)

    nct = (NCHUNK - wid + 31) // 32
    last = (nct - 1) % 2

    @pl.when(last == 0)
    def _():
        drain_outputs(0)

    @pl.when(last == 1)
    def _():
        drain_outputs(1)

    plsc.subcore_barrier()

    def out_body(ci, _):
        b = sid + ci * 16

        @pl.when(b < NBLK)
        def _():
            src = acc_sh.at[pl.ds(b * ZR, ZR), :]

            @pl.when(cid == 0)
            def _():
                pltpu.sync_copy(src, part0_hbm.at[pl.ds(b * ZR, ZR), :])

            @pl.when(cid == 1)
            def _():
                pltpu.sync_copy(src, part1_hbm.at[pl.ds(b * ZR, ZR), :])
        return 0

    lax.fori_loop(0, (NBLK + 15) // 16, out_body, 0)

    @pl.when(sid == 15)
    def _():
        src = acc_sh.at[pl.ds(NBLK * ZR, NTAIL), :]

        @pl.when(cid == 0)
        def _():
            pltpu.sync_copy(src, part0_hbm.at[pl.ds(NBLK * ZR, NTAIL), :])

        @pl.when(cid == 1)
        def _():
            pltpu.sync_copy(src, part1_hbm.at[pl.ds(NBLK * ZR, NTAIL), :])


_pass1 = functools.partial(
    pl.kernel,
    out_type=(jax.ShapeDtypeStruct((N_EDGES, 8), jnp.float32),
              jax.ShapeDtypeStruct((N_NODES, 8), jnp.float32),
              jax.ShapeDtypeStruct((N_NODES, 8), jnp.float32)),
    mesh=_MESH,
    compiler_params=_PARAMS,
    scratch_types=[
        pltpu.VMEM((2, CH), jnp.int32),
        pltpu.VMEM((2, CH), jnp.int32),
        pltpu.VMEM((C, 16), jnp.float32),
        pltpu.VMEM((C, 16), jnp.float32),
        pltpu.VMEM((C, 48), jnp.float32),
        pltpu.VMEM((C, 48), jnp.float32),
        pltpu.VMEM((C, 16), jnp.float32),
        pltpu.VMEM((C, 16), jnp.float32),
        pltpu.VMEM((C, 48), jnp.float32),
        pltpu.VMEM((C, 48), jnp.float32),
        pltpu.VMEM((C, 8), jnp.float32),
        pltpu.VMEM((C, 8), jnp.float32),
        pltpu.VMEM((ZR, 8), jnp.float32),
        pltpu.VMEM_SHARED((N_NODES, 8), jnp.float32),
        pltpu.SemaphoreType.DMA,
        pltpu.SemaphoreType.DMA,
        pltpu.SemaphoreType.DMA,
        pltpu.SemaphoreType.DMA,
        pltpu.SemaphoreType.DMA,
        pltpu.SemaphoreType.DMA,
        pltpu.SemaphoreType.DMA,
        pltpu.SemaphoreType.DMA,
    ],
)(_pass1_body)


def _merge_body(p0_ref, p1_ref, out_ref):
    out_ref[...] = p0_ref[...] + p1_ref[...]


def _merge(p0, p1):
    r = pl.pallas_call(
        _merge_body,
        out_shape=jax.ShapeDtypeStruct((N_NODES * 8 // 128, 128), jnp.float32),
    )(p0.reshape(-1, 128), p1.reshape(-1, 128))
    return r.reshape(N_NODES, 8)


_HEADS_A0 = [c // 2 for c in range(16)]
_HEADS_B0 = [c // 6 for c in range(16)]
_HEADS_A1 = [(16 + c) // 6 for c in range(16)]
_HEADS_B1 = [(32 + c) // 6 for c in range(16)]


def _pass2_compute(rows, ee_v, s_v, va_v, vb_v, m_v, heads_a, heads_b):
    used = sorted(set(heads_a) | set(heads_b))
    ah = {}
    for h in used:
        ah[h] = (plsc.load_gather(ee_v, [rows, _full(h)])
                 / plsc.load_gather(s_v, [rows, _full(h)]))
    for c in range(16):
        mv = ah[heads_a[c]] * plsc.load_gather(va_v, [rows, _full(c)])
        plsc.store_scatter(m_v, [rows, _full(c)], mv)
    for c in range(16):
        mv = ah[heads_b[c]] * plsc.load_gather(vb_v, [rows, _full(c)])
        plsc.store_scatter(m_v, [rows, _full(16 + c)], mv)


def _pass2_body(v0_hbm, v1_hbm, ee_hbm, ssum_hbm, dst_hbm,
                out0_hbm, out1_hbm,
                dst_v0, dst_v1, ee_v, s_v, va_v0, va_v1, vb_v0, vb_v1,
                m_v, z_v, acc_sh,
                semD0, semD1, semL0, semL1, semG, semE):
    dst_v = [dst_v0, dst_v1]
    va_v = [va_v0, va_v1]
    vb_v = [vb_v0, vb_v1]
    semD = [semD0, semD1]
    semL = [semL0, semL1]

    cid = lax.axis_index("c")
    sid = lax.axis_index("s")

    _zero_shared(z_v, acc_sh, sid, 32, 32)
    plsc.subcore_barrier()

    iota = lax.iota(jnp.int32, 16)

    def issue_loads(chunk, s):
        off = chunk * C

        @pl.when(cid == 0)
        def _():
            pltpu.async_copy(v0_hbm.at[pl.ds(off, C), :], va_v[s], semL[s])
            pltpu.async_copy(v1_hbm.at[pl.ds(off, C), pl.ds(0, 16)], vb_v[s],
                             semL[s])

        @pl.when(cid == 1)
        def _():
            pltpu.async_copy(v1_hbm.at[pl.ds(off, C), pl.ds(16, 16)], va_v[s],
                             semL[s])
            pltpu.async_copy(v1_hbm.at[pl.ds(off, C), pl.ds(32, 16)], vb_v[s],
                             semL[s])

    def wait_loads(s):
        pltpu.make_async_copy(v0_hbm.at[pl.ds(0, C), :], va_v[s], semL[s]).wait()
        pltpu.make_async_copy(v1_hbm.at[pl.ds(0, C), pl.ds(0, 16)], vb_v[s],
                              semL[s]).wait()

    def compute(s):
        @pl.when(cid == 0)
        def _():
            def gbody(g, _):
                _pass2_compute(iota + g * 16, ee_v, s_v, va_v[s],
                               vb_v[s], m_v, _HEADS_A0, _HEADS_B0)
                return 0
            lax.fori_loop(0, GRP, gbody, 0)

        @pl.when(cid == 1)
        def _():
            def gbody(g, _):
                _pass2_compute(iota + g * 16, ee_v, s_v, va_v[s],
                               vb_v[s], m_v, _HEADS_A1, _HEADS_B1)
                return 0
            lax.fori_loop(0, GRP, gbody, 0)

    pltpu.sync_copy(dst_hbm.at[pl.ds(sid * 2, 2), :], dst_v[0])
    issue_loads(sid, 0)

    def pair_body(cp, _):
        for b in (0, 1):
            chunk = sid + (2 * cp + b) * 16
            nchunk = chunk + 16

            @pl.when(chunk < NCHUNK)
            def _():
                @pl.when(nchunk < NCHUNK)
                def _():
                    pltpu.async_copy(dst_hbm.at[pl.ds(nchunk * 2, 2), :],
                                     dst_v[1 - b], semD[1 - b])

                ce = pltpu.async_copy(ee_hbm.at[pl.ds(chunk * C, C), :],
                                      ee_v, semE)
                cps = []
                for hh in range(2):
                    rr = pl.ds(hh * CH, CH)
                    cps.append(pltpu.async_copy(
                        ssum_hbm.at[dst_v[b].at[hh]], s_v.at[rr, :], semG))
                ce.wait()
                for cp2 in cps:
                    cp2.wait()
                wait_loads(b)
                compute(b)
                pltpu.sync_copy(m_v.at[pl.ds(0, CH), :],
                                acc_sh.at[dst_v[b].at[0]], add=True)
                pltpu.sync_copy(m_v.at[pl.ds(CH, CH), :],
                                acc_sh.at[dst_v[b].at[1]], add=True)

                @pl.when(nchunk < NCHUNK)
                def _():
                    pltpu.make_async_copy(dst_hbm.at[pl.ds(0, 2), :],
                                          dst_v[1 - b], semD[1 - b]).wait()
                    issue_loads(nchunk, 1 - b)
        return 0

    npairs = ((NCHUNK + 15) // 16 + 1) // 2
    lax.fori_loop(0, npairs, pair_body, 0)

    plsc.subcore_barrier()

    def out_body(ci, _):
        b = sid + ci * 16

        @pl.when(b < NBLK)
        def _():
            _copy_out_rows(acc_sh, out0_hbm, out1_hbm, cid, b * ZR, ZR)
        return 0

    lax.fori_loop(0, (NBLK + 15) // 16, out_body, 0)

    @pl.when(sid == 15)
    def _():
        _copy_out_rows(acc_sh, out0_hbm, out1_hbm, cid, NBLK * ZR, NTAIL)


def _copy_out_rows(acc_sh, out0_hbm, out1_hbm, cid, r0, nr):
    rows = pl.ds(r0, nr)

    @pl.when(cid == 0)
    def _():
        pltpu.sync_copy(acc_sh.at[rows, pl.ds(0, 16)], out0_hbm.at[rows, :])
        pltpu.sync_copy(acc_sh.at[rows, pl.ds(16, 16)],
                        out1_hbm.at[rows, pl.ds(0, 16)])

    @pl.when(cid == 1)
    def _():
        pltpu.sync_copy(acc_sh.at[rows, pl.ds(0, 16)],
                        out1_hbm.at[rows, pl.ds(16, 16)])
        pltpu.sync_copy(acc_sh.at[rows, pl.ds(16, 16)],
                        out1_hbm.at[rows, pl.ds(32, 16)])


_pass2 = functools.partial(
    pl.kernel,
    out_type=(jax.ShapeDtypeStruct((N_NODES, 16), jnp.float32),
              jax.ShapeDtypeStruct((N_NODES, 48), jnp.float32)),
    mesh=_MESH,
    compiler_params=_PARAMS,
    scratch_types=[
        pltpu.VMEM((2, CH), jnp.int32),
        pltpu.VMEM((2, CH), jnp.int32),
        pltpu.VMEM((C, 8), jnp.float32),
        pltpu.VMEM((C, 8), jnp.float32),
        pltpu.VMEM((C, 16), jnp.float32),
        pltpu.VMEM((C, 16), jnp.float32),
        pltpu.VMEM((C, 16), jnp.float32),
        pltpu.VMEM((C, 16), jnp.float32),
        pltpu.VMEM((C, 32), jnp.float32),
        pltpu.VMEM((32, 32), jnp.float32),
        pltpu.VMEM_SHARED((N_NODES, 32), jnp.float32),
        pltpu.SemaphoreType.DMA,
        pltpu.SemaphoreType.DMA,
        pltpu.SemaphoreType.DMA,
        pltpu.SemaphoreType.DMA,
        pltpu.SemaphoreType.DMA,
        pltpu.SemaphoreType.DMA,
    ],
)(_pass2_body)


def kernel(v0, v1, k0, k1, q0, q1, edge_index):
    E, N = N_EDGES, N_NODES
    dst = edge_index[1]
    k0f = k0.reshape(E, 16)
    k1f = k1.reshape(E, 48)
    q0f = q0.reshape(N, 16)
    q1f = q1.reshape(N, 48)
    v0f = v0.reshape(E, 16)
    v1f = v1.reshape(E, 48)

    dst2 = dst.reshape(E // 128, 128)
    ee, part0, part1 = _pass1(k0f, k1f, q0f, q1f, dst2)
    ssum = _merge(part0, part1)
    out0f, out1f = _pass2(v0f, v1f, ee, ssum, dst2)
    return out0f.reshape(N, 16, 1), out1f.reshape(N, 16, 3)
